# baseline (speedup 1.0000x reference)
import functools

import jax
import jax.numpy as jnp
from jax import lax
from jax.experimental import pallas as pl
from jax.experimental.pallas import tpu as pltpu
from jax.experimental.pallas import tpu_sc as plsc


NUM_ROWS = 100000
EMBED = 64
FLAT = NUM_ROWS * EMBED
NUM_CORES = 2
NUM_SUBCORES = 16
NUM_WORKERS = NUM_CORES * NUM_SUBCORES
PER_WORKER = FLAT // NUM_WORKERS
CHUNK = 50_000
NUM_CHUNKS = PER_WORKER // CHUNK


@functools.partial(
    pl.kernel,
    mesh=plsc.VectorSubcoreMesh(core_axis_name="c", subcore_axis_name="s"),
    out_type=jax.ShapeDtypeStruct((FLAT,), jnp.float32),
    scratch_types=[
        pltpu.VMEM((CHUNK,), jnp.float32),
        pltpu.VMEM((CHUNK,), jnp.float32),
        pltpu.SemaphoreType.DMA,
        pltpu.SemaphoreType.DMA,
        pltpu.SemaphoreType.DMA,
        pltpu.SemaphoreType.DMA,
    ],
)
def _sc_copy(in_hbm, out_hbm, buf0, buf1, sem0, sem1, osem0, osem1):
    wid = lax.axis_index("s") * NUM_CORES + lax.axis_index("c")
    base = wid * PER_WORKER
    bufs = (buf0, buf1)
    isems = (sem0, sem1)
    osems = (osem0, osem1)

    in_cp = []
    out_cp = []
    for i in range(NUM_CHUNKS):
        sl = pl.ds(base + i * CHUNK, CHUNK)
        b = bufs[i % 2]
        in_cp.append(pltpu.make_async_copy(in_hbm.at[sl], b, isems[i % 2]))
        out_cp.append(pltpu.make_async_copy(b, out_hbm.at[sl], osems[i % 2]))

    in_cp[0].start()
    in_cp[1].start()
    for i in range(NUM_CHUNKS):
        in_cp[i].wait()
        out_cp[i].start()
        out_cp[i].wait()
        if i + 2 < NUM_CHUNKS:
            in_cp[i + 2].start()


def kernel(x, W):
    del x
    flat = W.T.reshape(-1)
    out_flat = _sc_copy(flat)
    return out_flat.reshape(EMBED, NUM_ROWS).T


# speedup vs baseline: 1.0089x; 1.0089x over previous
import functools

import jax
import jax.numpy as jnp
from jax import lax
from jax.experimental import pallas as pl
from jax.experimental.pallas import tpu as pltpu
from jax.experimental.pallas import tpu_sc as plsc


NUM_ROWS = 100000
EMBED = 64
FLAT = NUM_ROWS * EMBED
NUM_CORES = 2
NUM_SUBCORES = 16
NUM_WORKERS = NUM_CORES * NUM_SUBCORES
PER_WORKER = FLAT // NUM_WORKERS
NBUF = 3
CHUNK = 40_000
NUM_CHUNKS = PER_WORKER // CHUNK


@functools.partial(
    pl.kernel,
    mesh=plsc.VectorSubcoreMesh(core_axis_name="c", subcore_axis_name="s"),
    out_type=jax.ShapeDtypeStruct((FLAT,), jnp.float32),
    scratch_types=[
        pltpu.VMEM((CHUNK,), jnp.float32),
        pltpu.VMEM((CHUNK,), jnp.float32),
        pltpu.VMEM((CHUNK,), jnp.float32),
        pltpu.SemaphoreType.DMA,
        pltpu.SemaphoreType.DMA,
        pltpu.SemaphoreType.DMA,
        pltpu.SemaphoreType.DMA,
        pltpu.SemaphoreType.DMA,
        pltpu.SemaphoreType.DMA,
    ],
)
def _sc_copy(in_hbm, out_hbm, b0, b1, b2, is0, is1, is2, os0, os1, os2):
    wid = lax.axis_index("s") * NUM_CORES + lax.axis_index("c")
    base = wid * PER_WORKER
    bufs = (b0, b1, b2)
    isems = (is0, is1, is2)
    osems = (os0, os1, os2)

    in_cp = []
    out_cp = []
    for i in range(NUM_CHUNKS):
        sl = pl.ds(base + i * CHUNK, CHUNK)
        b = bufs[i % NBUF]
        in_cp.append(pltpu.make_async_copy(in_hbm.at[sl], b, isems[i % NBUF]))
        out_cp.append(pltpu.make_async_copy(b, out_hbm.at[sl], osems[i % NBUF]))

    for i in range(min(NBUF, NUM_CHUNKS)):
        in_cp[i].start()
    for i in range(NUM_CHUNKS):
        in_cp[i].wait()
        out_cp[i].start()
        if i + NBUF < NUM_CHUNKS:
            out_cp[i].wait()
            in_cp[i + NBUF].start()
    for i in range(max(0, NUM_CHUNKS - NBUF), NUM_CHUNKS):
        out_cp[i].wait()


def kernel(x, W):
    del x
    flat = W.T.reshape(-1)
    out_flat = _sc_copy(flat)
    return out_flat.reshape(EMBED, NUM_ROWS).T
